# Initial kernel scaffold; baseline (speedup 1.0000x reference)
import jax
import jax.numpy as jnp
from jax.experimental import pallas as pl


def kernel(scores):
    raise NotImplementedError("write your pallas kernel here")


# speedup vs baseline: 2.0461x; 2.0461x over previous
import functools

import jax
import jax.numpy as jnp
from jax import lax
from jax.experimental import pallas as pl
from jax.experimental.pallas import tpu as pltpu

K_TOP = 10
ROWS_PER_BLOCK = 8


def _topk_mask_block(x_ref, o_ref):
    x = x_ref[...]
    R, N = x.shape
    neg = jnp.float32(-jnp.inf)

    work = x
    t = jnp.full((R, 1), neg, jnp.float32)
    cum = jnp.zeros((R, 1), jnp.int32)
    for _ in range(K_TOP):
        m = jnp.max(work, axis=-1, keepdims=True)
        eq = work == m
        c = jnp.sum(eq.astype(jnp.int32), axis=-1, keepdims=True)
        take = cum < K_TOP
        t = jnp.where(take, m, t)
        cum = cum + c
        work = jnp.where(eq, neg, work)

    ge = x >= t
    n_ge = jnp.sum(ge.astype(jnp.int32), axis=-1, keepdims=True)
    exact = jnp.all(n_ge == K_TOP)

    def fast(_):
        return ge.astype(jnp.float32)

    def slow(_):
        gt = x > t
        g = jnp.sum(gt.astype(jnp.int32), axis=-1, keepdims=True)
        eqm = jnp.logical_and(x == t, jnp.logical_not(gt))
        iota = lax.broadcasted_iota(jnp.int32, (R, N), 1)
        big = jnp.int32(N + 1)
        sel = jnp.zeros((R, N), jnp.bool_)
        taken = g
        for _ in range(K_TOP):
            cand = jnp.where(jnp.logical_and(eqm, jnp.logical_not(sel)), iota, big)
            idx = jnp.min(cand, axis=-1, keepdims=True)
            add = jnp.logical_and(taken < K_TOP, idx < big)
            sel = jnp.logical_or(sel, jnp.logical_and(iota == idx, add))
            taken = taken + add.astype(jnp.int32)
        return jnp.logical_or(gt, sel).astype(jnp.float32)

    o_ref[...] = lax.cond(exact, fast, slow, 0)


@jax.jit
def kernel(scores):
    B, N = scores.shape
    grid = (B // ROWS_PER_BLOCK,)
    return pl.pallas_call(
        _topk_mask_block,
        grid=grid,
        in_specs=[pl.BlockSpec((ROWS_PER_BLOCK, N), lambda i: (i, 0))],
        out_specs=pl.BlockSpec((ROWS_PER_BLOCK, N), lambda i: (i, 0)),
        out_shape=jax.ShapeDtypeStruct((B, N), jnp.float32),
        compiler_params=pltpu.CompilerParams(
            dimension_semantics=("arbitrary",),
        ),
    )(scores)


# speedup vs baseline: 3.8833x; 1.8980x over previous
import functools

import jax
import jax.numpy as jnp
from jax import lax
from jax.experimental import pallas as pl
from jax.experimental.pallas import tpu as pltpu

K_TOP = 10
ROWS_PER_BLOCK = 8


def _topk_mask_block(x_ref, o_ref):
    R, N = x_ref.shape
    LANES = 128
    n_chunks = N // LANES
    neg = jnp.float32(-jnp.inf)

    tops = [jnp.full((R, LANES), neg, jnp.float32) for _ in range(K_TOP)]
    for c in range(n_chunks):
        v = x_ref[:, c * LANES:(c + 1) * LANES]
        for i in range(K_TOP):
            hi = jnp.maximum(tops[i], v)
            v = jnp.minimum(tops[i], v)
            tops[i] = hi

    t = jnp.full((R, 1), neg, jnp.float32)
    cum = jnp.zeros((R, 1), jnp.int32)
    for _ in range(K_TOP):
        mv = tops[0]
        for i in range(1, K_TOP):
            mv = jnp.maximum(mv, tops[i])
        m = jnp.max(mv, axis=-1, keepdims=True)
        cnt = jnp.zeros((R, LANES), jnp.int32)
        for i in range(K_TOP):
            cnt = cnt + (tops[i] == m).astype(jnp.int32)
        c = jnp.sum(cnt, axis=-1, keepdims=True)
        take = cum < K_TOP
        t = jnp.where(take, m, t)
        cum = cum + c
        for i in range(K_TOP):
            tops[i] = jnp.where(tops[i] == m, neg, tops[i])

    x = x_ref[...]
    ge = x >= t
    n_ge = jnp.sum(ge.astype(jnp.int32), axis=-1, keepdims=True)
    exact = jnp.all(n_ge == K_TOP)

    def fast(_):
        return ge.astype(jnp.float32)

    def slow(_):
        gt = x > t
        g = jnp.sum(gt.astype(jnp.int32), axis=-1, keepdims=True)
        eqm = jnp.logical_and(x == t, jnp.logical_not(gt))
        iota = lax.broadcasted_iota(jnp.int32, (R, N), 1)
        big = jnp.int32(N + 1)
        sel = jnp.zeros((R, N), jnp.bool_)
        taken = g
        for _ in range(K_TOP):
            cand = jnp.where(jnp.logical_and(eqm, jnp.logical_not(sel)), iota, big)
            idx = jnp.min(cand, axis=-1, keepdims=True)
            add = jnp.logical_and(taken < K_TOP, idx < big)
            sel = jnp.logical_or(sel, jnp.logical_and(iota == idx, add))
            taken = taken + add.astype(jnp.int32)
        return jnp.logical_or(gt, sel).astype(jnp.float32)

    o_ref[...] = lax.cond(exact, fast, slow, 0)


@jax.jit
def kernel(scores):
    B, N = scores.shape
    grid = (B // ROWS_PER_BLOCK,)
    return pl.pallas_call(
        _topk_mask_block,
        grid=grid,
        in_specs=[pl.BlockSpec((ROWS_PER_BLOCK, N), lambda i: (i, 0))],
        out_specs=pl.BlockSpec((ROWS_PER_BLOCK, N), lambda i: (i, 0)),
        out_shape=jax.ShapeDtypeStruct((B, N), jnp.float32),
        compiler_params=pltpu.CompilerParams(
            dimension_semantics=("arbitrary",),
        ),
    )(scores)


# speedup vs baseline: 4.3364x; 1.1167x over previous
import functools

import jax
import jax.numpy as jnp
from jax import lax
from jax.experimental import pallas as pl
from jax.experimental.pallas import tpu as pltpu

K_TOP = 10
ROWS_PER_BLOCK = 8


def _topk_mask_block(x_ref, o_ref):
    R, N = x_ref.shape
    LANES = 128
    n_chunks = N // LANES
    neg = jnp.float32(-jnp.inf)

    CAPS = (K_TOP, 5, 3, 2)
    G = len(CAPS)
    n_groups = n_chunks // G
    tops = [[jnp.full((R, LANES), neg, jnp.float32) for _ in range(cap)]
            for cap in CAPS]

    def ce(a, b):
        return jnp.maximum(a, b), jnp.minimum(a, b)

    for g in range(n_groups):
        base = g * G * LANES
        v0 = x_ref[:, base:base + LANES]
        v1 = x_ref[:, base + LANES:base + 2 * LANES]
        v2 = x_ref[:, base + 2 * LANES:base + 3 * LANES]
        v3 = x_ref[:, base + 3 * LANES:base + 4 * LANES]
        a, b = ce(v0, v1)
        c_, d = ce(v2, v3)
        s0, c2 = ce(a, c_)
        b2, s3 = ce(b, d)
        s1, s2 = ce(b2, c2)
        for s, v in enumerate((s0, s1, s2, s3)):
            ts = tops[s]
            for i in range(CAPS[s]):
                hi = jnp.maximum(ts[i], v)
                v = jnp.minimum(ts[i], v)
                ts[i] = hi

    cand = jnp.concatenate([t for ts in tops for t in ts], axis=-1)
    t = jnp.full((R, 1), neg, jnp.float32)
    cum = jnp.zeros((R, 1), jnp.int32)
    for _ in range(K_TOP):
        m = jnp.max(cand, axis=-1, keepdims=True)
        eq = cand == m
        c = jnp.sum(eq.astype(jnp.int32), axis=-1, keepdims=True)
        take = cum < K_TOP
        t = jnp.where(take, m, t)
        cum = cum + c
        cand = jnp.where(eq, neg, cand)

    x = x_ref[...]
    ge = x >= t
    n_ge = jnp.sum(ge.astype(jnp.int32), axis=-1, keepdims=True)
    exact = jnp.all(n_ge == K_TOP)

    def fast(_):
        return ge.astype(jnp.float32)

    def slow(_):
        gt = x > t
        g = jnp.sum(gt.astype(jnp.int32), axis=-1, keepdims=True)
        eqm = jnp.logical_and(x == t, jnp.logical_not(gt))
        iota = lax.broadcasted_iota(jnp.int32, (R, N), 1)
        big = jnp.int32(N + 1)
        sel = jnp.zeros((R, N), jnp.bool_)
        taken = g
        for _ in range(K_TOP):
            cand = jnp.where(jnp.logical_and(eqm, jnp.logical_not(sel)), iota, big)
            idx = jnp.min(cand, axis=-1, keepdims=True)
            add = jnp.logical_and(taken < K_TOP, idx < big)
            sel = jnp.logical_or(sel, jnp.logical_and(iota == idx, add))
            taken = taken + add.astype(jnp.int32)
        return jnp.logical_or(gt, sel).astype(jnp.float32)

    o_ref[...] = lax.cond(exact, fast, slow, 0)


@jax.jit
def kernel(scores):
    B, N = scores.shape
    grid = (B // ROWS_PER_BLOCK,)
    return pl.pallas_call(
        _topk_mask_block,
        grid=grid,
        in_specs=[pl.BlockSpec((ROWS_PER_BLOCK, N), lambda i: (i, 0))],
        out_specs=pl.BlockSpec((ROWS_PER_BLOCK, N), lambda i: (i, 0)),
        out_shape=jax.ShapeDtypeStruct((B, N), jnp.float32),
        compiler_params=pltpu.CompilerParams(
            dimension_semantics=("arbitrary",),
        ),
    )(scores)
